# baseline (speedup 1.0000x reference)
import functools

import jax
import jax.numpy as jnp
from jax import lax
from jax.experimental import pallas as pl
from jax.experimental.pallas import tpu as pltpu

_K = 40
_INF_BITS = 0x7F800000


def _elu(x):
    return jnp.where(x > 0, x, jnp.exp(x) - 1.0)


def _enc_body(x_ref, W1_ref, b1_ref, W2_ref, b2_ref, W3_ref, b3_ref,
              Ws_ref, bs_ref, Wh_ref, bh_ref,
              xenc_ref, s_ref, h_ref):
    x = x_ref[:, :]
    a = _elu(jnp.dot(x, W1_ref[:, :], preferred_element_type=jnp.float32)
             + b1_ref[:, :])
    a = _elu(jnp.dot(a, W2_ref[:, :], preferred_element_type=jnp.float32)
             + b2_ref[:, :])
    xe = jnp.dot(a, W3_ref[:, :], preferred_element_type=jnp.float32) + b3_ref[:, :]
    xenc_ref[:, :] = xe
    s_ref[:, :] = jnp.dot(xe, Ws_ref[:, :], preferred_element_type=jnp.float32) + bs_ref[:, :]
    h_ref[:, :] = jnp.dot(xe, Wh_ref[:, :], preferred_element_type=jnp.float32) + bh_ref[:, :]


def _grav_body(C, bounds_ref, s_row_ref, xenc_ref, xpfc_ref, batr_ref,
               sT_ref, hT_ref, h_ref, batc_ref,
               lo1_ref, lo2_ref, lo2b_ref, f1_ref, f1b_ref, f2_ref, f2b_ref,
               o1_ref, o1b_ref, o2_ref, o2b_ref,
               out_ref, d2_ref):
    R = s_row_ref.shape[0]
    P = h_ref.shape[1]
    i = pl.program_id(0)
    c_lo = bounds_ref[i, 0]
    c_hi = bounds_ref[i, 1]

    s_r = s_row_ref[:, :]
    sq_r = jnp.sum(s_r * s_r, axis=1, keepdims=True)
    bat_r = batr_ref[:, :]

    def fill(c, _):
        sT_c = sT_ref[:, pl.ds(c * C, C)]
        sq_c = jnp.sum(sT_c * sT_c, axis=0, keepdims=True)
        d2 = sq_r + sq_c - 2.0 * jnp.dot(
            s_r, sT_c, preferred_element_type=jnp.float32)
        bat_c = batc_ref[:, pl.ds(c * C, C)]
        d2 = jnp.where(bat_r == bat_c, d2, jnp.inf)
        d2_ref[:, pl.ds(c * C, C)] = jnp.maximum(d2, 0.0)
        return 0

    lax.fori_loop(c_lo, c_hi, fill, 0)

    L = 128

    def _lgroups(m):
        acc = m[:, :L]
        for g in range(1, C // L):
            acc = acc + m[:, g * L:(g + 1) * L]
        return acc

    def count_le(t_f):
        def cbody(c, acc):
            d2 = d2_ref[:, pl.ds(c * C, C)]
            return acc + _lgroups(jnp.where(d2 <= t_f, 1.0, 0.0))
        acc = lax.fori_loop(c_lo, c_hi, cbody,
                            jnp.zeros((R, L), jnp.float32))
        return jnp.sum(acc, axis=1, keepdims=True)

    def scond(state):
        it, lo, hi, tau, ndone = state
        return jnp.logical_and(it < 26, ndone < float(R))

    def sbody(state):
        it, lo, hi, tau, ndone = state
        mid = lo + (hi - lo) // 2
        mid_f = lax.bitcast_convert_type(mid, jnp.float32)
        cnt = count_le(mid_f)
        done = jnp.logical_not(jnp.isinf(tau))
        hit = jnp.logical_and(cnt == float(_K), jnp.logical_not(done))
        tau = jnp.where(hit, mid_f, tau)
        take = cnt >= float(_K)
        lo = jnp.where(done, lo, jnp.where(take, lo, mid))
        hi = jnp.where(done, hi, jnp.where(take, mid, hi))
        ndone = jnp.sum(jnp.where(jnp.logical_or(hit, done), 1.0, 0.0))
        return (it + 1, lo, hi, tau, ndone)

    lo0 = jnp.full((R, 1), -1, jnp.int32)
    hi0 = jnp.full((R, 1), _INF_BITS, jnp.int32)
    tau0 = jnp.full((R, 1), jnp.inf, jnp.float32)
    _, _, hi, tau, _ = lax.while_loop(
        scond, sbody, (jnp.int32(0), lo0, hi0, tau0, jnp.float32(0.0)))
    t_f = jnp.where(jnp.isinf(tau),
                    lax.bitcast_convert_type(hi, jnp.float32), tau)

    def agg_body(c, carry):
        s_lt, s_eq, max_acc, n_lt, n_eq = carry
        d2 = d2_ref[:, pl.ds(c * C, C)]
        w = jnp.exp(-10.0 * d2)
        lt = d2 < t_f
        eq = d2 == t_f
        h_c = h_ref[pl.ds(c * C, C), :]
        w_lt = jnp.where(lt, w, 0.0)
        w_eq = jnp.where(eq, w, 0.0)
        s_lt = s_lt + jnp.dot(w_lt, h_c, preferred_element_type=jnp.float32)
        s_eq = s_eq + jnp.dot(w_eq, h_c, preferred_element_type=jnp.float32)
        n_lt = n_lt + _lgroups(jnp.where(lt, 1.0, 0.0))
        n_eq = n_eq + _lgroups(jnp.where(eq, 1.0, 0.0))
        sel = lt | eq
        new_max = []
        for p in range(P):
            h_p = hT_ref[p:p + 1, pl.ds(c * C, C)]
            prod = jnp.where(sel, w * h_p, -jnp.inf)
            m = jnp.maximum(prod[:, :L], prod[:, L:2 * L])
            for g in range(2, C // L):
                m = jnp.maximum(m, prod[:, g * L:(g + 1) * L])
            new_max.append(jnp.maximum(max_acc[p], m))
        return (s_lt, s_eq, tuple(new_max), n_lt, n_eq)

    zf = jnp.zeros((R, P), jnp.float32)
    zL = jnp.zeros((R, L), jnp.float32)
    max0 = tuple(jnp.full((R, L), -jnp.inf, jnp.float32) for _ in range(P))
    s_lt, s_eq, max_acc, n_lt, n_eq = lax.fori_loop(
        c_lo, c_hi, agg_body, (zf, zf, max0, zL, zL))

    n_lt = jnp.sum(n_lt, axis=1, keepdims=True)
    n_eq = jnp.sum(n_eq, axis=1, keepdims=True)
    frac = (float(_K) - n_lt) / jnp.maximum(n_eq, 1.0)
    maxs = jnp.concatenate(
        [jnp.max(m, axis=1, keepdims=True) for m in max_acc], axis=1)
    agg = jnp.concatenate([(s_lt + frac * s_eq) / float(_K), maxs],
                          axis=1)

    xe = xenc_ref[:, :]
    feats = (jnp.dot(xe, lo1_ref[:, :], preferred_element_type=jnp.float32)
             + jnp.dot(agg, lo2_ref[:, :], preferred_element_type=jnp.float32)
             + lo2b_ref[:, :])
    mu = jnp.mean(feats, axis=1, keepdims=True)
    d = feats - mu
    var = jnp.mean(d * d, axis=1, keepdims=True)
    f = d / jnp.sqrt(var + 1e-5)

    H = f.shape[1]
    W1s = f1_ref[:H, :] + f1_ref[H:, :]
    y = _elu(jnp.dot(f, W1s, preferred_element_type=jnp.float32) + f1b_ref[:, :])
    y = jnp.dot(y, f2_ref[:, :], preferred_element_type=jnp.float32) + f2b_ref[:, :]

    ncat = y.shape[1]
    z = (jnp.dot(y, o1_ref[:ncat, :], preferred_element_type=jnp.float32)
         + jnp.dot(xpfc_ref[:, :], o1_ref[ncat:, :],
                   preferred_element_type=jnp.float32)
         + o1b_ref[:, :])
    z = _elu(z)
    out_ref[:, :] = (jnp.dot(z, o2_ref[:, :], preferred_element_type=jnp.float32)
                     + o2b_ref[:, :])


def kernel(x_pfc, batch_pfc, enc_W1, enc_b1, enc_W2, enc_b2, enc_W3, enc_b3,
           lin_s_W, lin_s_b, lin_h_W, lin_h_b, lin_out1_W, lin_out2_W,
           lin_out2_b, ffn_W1, ffn_b1, ffn_W2, ffn_b2, out_W1, out_b1,
           out_W2, out_b2):
    N, PFC = x_pfc.shape
    S = lin_s_W.shape[1]
    P = lin_h_W.shape[1]
    H = enc_W3.shape[1]

    R = min(256, N)
    C = min(256, N)
    nblk = N // R

    row2 = lambda b: b.reshape(1, -1)

    xenc, s, h = pl.pallas_call(
        _enc_body,
        out_shape=(
            jax.ShapeDtypeStruct((N, H), jnp.float32),
            jax.ShapeDtypeStruct((N, S), jnp.float32),
            jax.ShapeDtypeStruct((N, P), jnp.float32),
        ),
    )(x_pfc, enc_W1, row2(enc_b1), enc_W2, row2(enc_b2), enc_W3, row2(enc_b3),
      lin_s_W, row2(lin_s_b), lin_h_W, row2(lin_h_b))

    sT = s.T
    hT = h.T
    bat = batch_pfc.astype(jnp.int32)
    bat_r = bat.reshape(N, 1)
    bat_c = bat.reshape(1, N)

    row_starts = jnp.arange(nblk, dtype=jnp.int32) * R
    b_lo = bat[row_starts]
    b_hi = bat[row_starts + (R - 1)]
    col_lo = jnp.searchsorted(bat, b_lo, side="left").astype(jnp.int32)
    col_hi = jnp.searchsorted(bat, b_hi, side="right").astype(jnp.int32)
    bounds = jnp.stack([col_lo // C, (col_hi + (C - 1)) // C], axis=1)

    full = lambda a: pl.BlockSpec(a.shape, lambda i, b: (0,) * a.ndim)
    rows = lambda ncol: pl.BlockSpec((R, ncol), lambda i, b: (i, 0))

    body = functools.partial(_grav_body, C)

    args = (s, xenc, x_pfc, bat_r, sT, hT, h, bat_c,
            lin_out1_W, lin_out2_W, row2(lin_out2_b),
            ffn_W1, row2(ffn_b1), ffn_W2, row2(ffn_b2),
            out_W1, row2(out_b1), out_W2, row2(out_b2))
    specs = [rows(S), rows(H), rows(PFC), rows(1),
             full(sT), full(hT), full(h), full(bat_c)] + \
            [full(a) for a in args[8:]]

    grid_spec = pltpu.PrefetchScalarGridSpec(
        num_scalar_prefetch=1,
        grid=(nblk,),
        in_specs=specs,
        out_specs=rows(1),
        scratch_shapes=[pltpu.VMEM((R, N), jnp.float32)],
    )

    out = pl.pallas_call(
        body,
        grid_spec=grid_spec,
        out_shape=jax.ShapeDtypeStruct((N, 1), jnp.float32),
        compiler_params=pltpu.CompilerParams(
            dimension_semantics=("arbitrary",)),
    )(bounds, *args)

    return (out, batch_pfc)


# speedup vs baseline: 1.2085x; 1.2085x over previous
import functools

import jax
import jax.numpy as jnp
from jax import lax
from jax.experimental import pallas as pl
from jax.experimental.pallas import tpu as pltpu

_K = 40
_INF_BITS = 0x7F800000


def _elu(x):
    return jnp.where(x > 0, x, jnp.exp(x) - 1.0)


def _enc_body(x_ref, W1_ref, b1_ref, W2_ref, b2_ref, W3_ref, b3_ref,
              Ws_ref, bs_ref, Wh_ref, bh_ref,
              xenc_ref, s_ref, h_ref):
    x = x_ref[:, :]
    a = _elu(jnp.dot(x, W1_ref[:, :], preferred_element_type=jnp.float32)
             + b1_ref[:, :])
    a = _elu(jnp.dot(a, W2_ref[:, :], preferred_element_type=jnp.float32)
             + b2_ref[:, :])
    xe = jnp.dot(a, W3_ref[:, :], preferred_element_type=jnp.float32) + b3_ref[:, :]
    xenc_ref[:, :] = xe
    s_ref[:, :] = jnp.dot(xe, Ws_ref[:, :], preferred_element_type=jnp.float32) + bs_ref[:, :]
    h_ref[:, :] = jnp.dot(xe, Wh_ref[:, :], preferred_element_type=jnp.float32) + bh_ref[:, :]


def _grav_body(C, bounds_ref, s_row_ref, xenc_ref, xpfc_ref, batr_ref,
               sT_ref, hT_ref, h_ref, batc_ref,
               lo1_ref, lo2_ref, lo2b_ref, f1_ref, f1b_ref, f2_ref, f2b_ref,
               o1_ref, o1b_ref, o2_ref, o2b_ref,
               out_ref, d2_ref):
    R = s_row_ref.shape[0]
    P = h_ref.shape[1]
    i = pl.program_id(0)
    c_lo = bounds_ref[i, 0]
    c_hi = bounds_ref[i, 1]

    s_r = s_row_ref[:, :]
    sq_r = jnp.sum(s_r * s_r, axis=1, keepdims=True)
    bat_r = batr_ref[:, :]

    def fill(c, _):
        sT_c = sT_ref[:, pl.ds(c * C, C)]
        sq_c = jnp.sum(sT_c * sT_c, axis=0, keepdims=True)
        d2 = sq_r + sq_c - 2.0 * jnp.dot(
            s_r, sT_c, preferred_element_type=jnp.float32)
        bat_c = batc_ref[:, pl.ds(c * C, C)]
        d2 = jnp.where(bat_r == bat_c, d2, jnp.inf)
        d2_ref[:, pl.ds(c * C, C)] = jnp.maximum(d2, 0.0)
        return 0

    lax.fori_loop(c_lo, c_hi, fill, 0)

    L = 128

    def _lgroups(m):
        acc = m[:, :L]
        for g in range(1, C // L):
            acc = acc + m[:, g * L:(g + 1) * L]
        return acc

    def count_le(t_f):
        def cbody(c, acc):
            d2 = d2_ref[:, pl.ds(c * C, C)]
            return acc + _lgroups(jnp.where(d2 <= t_f, 1.0, 0.0))
        acc = lax.fori_loop(c_lo, c_hi, cbody,
                            jnp.zeros((R, L), jnp.float32))
        return jnp.sum(acc, axis=1, keepdims=True)

    _CAP = 22

    def sbody(it, state):
        lo, hi, tau = state
        mid = lo + (hi - lo) // 2
        mid_f = lax.bitcast_convert_type(mid, jnp.float32)
        cnt = count_le(mid_f)
        done = jnp.logical_not(jnp.isinf(tau))
        hit = jnp.logical_and(cnt == float(_K), jnp.logical_not(done))
        tau = jnp.where(hit, mid_f, tau)
        take = cnt >= float(_K)
        lo = jnp.where(done, lo, jnp.where(take, lo, mid))
        hi = jnp.where(done, hi, jnp.where(take, mid, hi))
        return (lo, hi, tau)

    lo0 = jnp.full((R, 1), -1, jnp.int32)
    hi0 = jnp.full((R, 1), _INF_BITS, jnp.int32)
    tau0 = jnp.full((R, 1), jnp.inf, jnp.float32)
    lo, hi, tau = lax.fori_loop(0, _CAP, sbody, (lo0, hi0, tau0))
    hit_rows = jnp.logical_not(jnp.isinf(tau))
    a_f = jnp.where(hit_rows, tau,
                    lax.bitcast_convert_type(lo, jnp.float32))
    b_f = jnp.where(hit_rows, tau,
                    lax.bitcast_convert_type(hi, jnp.float32))

    def agg_body(c, carry):
        s_a, s_b, max_acc, n_a, n_b = carry
        d2 = d2_ref[:, pl.ds(c * C, C)]
        w = jnp.exp(-10.0 * d2)
        lt = d2 <= a_f
        le = d2 <= b_f
        h_c = h_ref[pl.ds(c * C, C), :]
        w_a = jnp.where(lt, w, 0.0)
        w_b = jnp.where(le, w, 0.0)
        s_a = s_a + jnp.dot(w_a, h_c, preferred_element_type=jnp.float32)
        s_b = s_b + jnp.dot(w_b, h_c, preferred_element_type=jnp.float32)
        n_a = n_a + _lgroups(jnp.where(lt, 1.0, 0.0))
        n_b = n_b + _lgroups(jnp.where(le, 1.0, 0.0))
        new_max = []
        for p in range(P):
            h_p = hT_ref[p:p + 1, pl.ds(c * C, C)]
            prod = jnp.where(le, w * h_p, -jnp.inf)
            m = jnp.maximum(prod[:, :L], prod[:, L:2 * L])
            for g in range(2, C // L):
                m = jnp.maximum(m, prod[:, g * L:(g + 1) * L])
            new_max.append(jnp.maximum(max_acc[p], m))
        return (s_a, s_b, tuple(new_max), n_a, n_b)

    zf = jnp.zeros((R, P), jnp.float32)
    zL = jnp.zeros((R, L), jnp.float32)
    max0 = tuple(jnp.full((R, L), -jnp.inf, jnp.float32) for _ in range(P))
    s_a, s_b, max_acc, n_a, n_b = lax.fori_loop(
        c_lo, c_hi, agg_body, (zf, zf, max0, zL, zL))

    n_a = jnp.sum(n_a, axis=1, keepdims=True)
    n_band = jnp.sum(n_b, axis=1, keepdims=True) - n_a
    frac = (float(_K) - n_a) / jnp.maximum(n_band, 1.0)
    maxs = jnp.concatenate(
        [jnp.max(m, axis=1, keepdims=True) for m in max_acc], axis=1)
    agg = jnp.concatenate([(s_a + frac * (s_b - s_a)) / float(_K), maxs],
                          axis=1)

    xe = xenc_ref[:, :]
    feats = (jnp.dot(xe, lo1_ref[:, :], preferred_element_type=jnp.float32)
             + jnp.dot(agg, lo2_ref[:, :], preferred_element_type=jnp.float32)
             + lo2b_ref[:, :])
    mu = jnp.mean(feats, axis=1, keepdims=True)
    d = feats - mu
    var = jnp.mean(d * d, axis=1, keepdims=True)
    f = d / jnp.sqrt(var + 1e-5)

    H = f.shape[1]
    W1s = f1_ref[:H, :] + f1_ref[H:, :]
    y = _elu(jnp.dot(f, W1s, preferred_element_type=jnp.float32) + f1b_ref[:, :])
    y = jnp.dot(y, f2_ref[:, :], preferred_element_type=jnp.float32) + f2b_ref[:, :]

    ncat = y.shape[1]
    z = (jnp.dot(y, o1_ref[:ncat, :], preferred_element_type=jnp.float32)
         + jnp.dot(xpfc_ref[:, :], o1_ref[ncat:, :],
                   preferred_element_type=jnp.float32)
         + o1b_ref[:, :])
    z = _elu(z)
    out_ref[:, :] = (jnp.dot(z, o2_ref[:, :], preferred_element_type=jnp.float32)
                     + o2b_ref[:, :])


def kernel(x_pfc, batch_pfc, enc_W1, enc_b1, enc_W2, enc_b2, enc_W3, enc_b3,
           lin_s_W, lin_s_b, lin_h_W, lin_h_b, lin_out1_W, lin_out2_W,
           lin_out2_b, ffn_W1, ffn_b1, ffn_W2, ffn_b2, out_W1, out_b1,
           out_W2, out_b2):
    N, PFC = x_pfc.shape
    S = lin_s_W.shape[1]
    P = lin_h_W.shape[1]
    H = enc_W3.shape[1]

    R = min(256, N)
    C = min(512, N)
    nblk = N // R

    row2 = lambda b: b.reshape(1, -1)

    xenc, s, h = pl.pallas_call(
        _enc_body,
        out_shape=(
            jax.ShapeDtypeStruct((N, H), jnp.float32),
            jax.ShapeDtypeStruct((N, S), jnp.float32),
            jax.ShapeDtypeStruct((N, P), jnp.float32),
        ),
    )(x_pfc, enc_W1, row2(enc_b1), enc_W2, row2(enc_b2), enc_W3, row2(enc_b3),
      lin_s_W, row2(lin_s_b), lin_h_W, row2(lin_h_b))

    sT = s.T
    hT = h.T
    bat = batch_pfc.astype(jnp.int32)
    bat_r = bat.reshape(N, 1)
    bat_c = bat.reshape(1, N)

    row_starts = jnp.arange(nblk, dtype=jnp.int32) * R
    b_lo = bat[row_starts]
    b_hi = bat[row_starts + (R - 1)]
    col_lo = jnp.searchsorted(bat, b_lo, side="left").astype(jnp.int32)
    col_hi = jnp.searchsorted(bat, b_hi, side="right").astype(jnp.int32)
    bounds = jnp.stack([col_lo // C, (col_hi + (C - 1)) // C], axis=1)

    full = lambda a: pl.BlockSpec(a.shape, lambda i, b: (0,) * a.ndim)
    rows = lambda ncol: pl.BlockSpec((R, ncol), lambda i, b: (i, 0))

    body = functools.partial(_grav_body, C)

    args = (s, xenc, x_pfc, bat_r, sT, hT, h, bat_c,
            lin_out1_W, lin_out2_W, row2(lin_out2_b),
            ffn_W1, row2(ffn_b1), ffn_W2, row2(ffn_b2),
            out_W1, row2(out_b1), out_W2, row2(out_b2))
    specs = [rows(S), rows(H), rows(PFC), rows(1),
             full(sT), full(hT), full(h), full(bat_c)] + \
            [full(a) for a in args[8:]]

    grid_spec = pltpu.PrefetchScalarGridSpec(
        num_scalar_prefetch=1,
        grid=(nblk,),
        in_specs=specs,
        out_specs=rows(1),
        scratch_shapes=[pltpu.VMEM((R, N), jnp.float32)],
    )

    out = pl.pallas_call(
        body,
        grid_spec=grid_spec,
        out_shape=jax.ShapeDtypeStruct((N, 1), jnp.float32),
        compiler_params=pltpu.CompilerParams(
            dimension_semantics=("arbitrary",)),
    )(bounds, *args)

    return (out, batch_pfc)


# speedup vs baseline: 1.2835x; 1.0621x over previous
import functools

import jax
import jax.numpy as jnp
from jax import lax
from jax.experimental import pallas as pl
from jax.experimental.pallas import tpu as pltpu

_K = 40
_INF_BITS = 0x7F800000


def _elu(x):
    return jnp.where(x > 0, x, jnp.exp(x) - 1.0)


def _enc_body(x_ref, W1_ref, b1_ref, W2_ref, b2_ref, W3_ref, b3_ref,
              Ws_ref, bs_ref, Wh_ref, bh_ref,
              xenc_ref, s_ref, h_ref):
    x = x_ref[:, :]
    a = _elu(jnp.dot(x, W1_ref[:, :], preferred_element_type=jnp.float32)
             + b1_ref[:, :])
    a = _elu(jnp.dot(a, W2_ref[:, :], preferred_element_type=jnp.float32)
             + b2_ref[:, :])
    xe = jnp.dot(a, W3_ref[:, :], preferred_element_type=jnp.float32) + b3_ref[:, :]
    xenc_ref[:, :] = xe
    s_ref[:, :] = jnp.dot(xe, Ws_ref[:, :], preferred_element_type=jnp.float32) + bs_ref[:, :]
    h_ref[:, :] = jnp.dot(xe, Wh_ref[:, :], preferred_element_type=jnp.float32) + bh_ref[:, :]


def _grav_body(C, bounds_ref, s_row_ref, xenc_ref, xpfc_ref, batr_ref,
               sT_ref, hT_ref, h_ref, batc_ref,
               lo1_ref, lo2_ref, lo2b_ref, f1_ref, f1b_ref, f2_ref, f2b_ref,
               o1_ref, o1b_ref, o2_ref, o2b_ref,
               out_ref, d2_ref):
    R = s_row_ref.shape[0]
    P = h_ref.shape[1]
    i = pl.program_id(0)
    c_lo = bounds_ref[i, 0]
    c_hi = bounds_ref[i, 1]

    s_r = s_row_ref[:, :]
    sq_r = jnp.sum(s_r * s_r, axis=1, keepdims=True)
    bat_r = batr_ref[:, :]

    def fill(c, _):
        sT_c = sT_ref[:, pl.ds(c * C, C)]
        sq_c = jnp.sum(sT_c * sT_c, axis=0, keepdims=True)
        d2 = sq_r + sq_c - 2.0 * jnp.dot(
            s_r, sT_c, preferred_element_type=jnp.float32)
        bat_c = batc_ref[:, pl.ds(c * C, C)]
        d2 = jnp.where(bat_r == bat_c, d2, jnp.inf)
        d2_ref[:, pl.ds(c * C, C)] = jnp.maximum(d2, 0.0)
        return 0

    lax.fori_loop(c_lo, c_hi, fill, 0)

    L = 128

    def _lgroups(m):
        acc = m[:, :L]
        for g in range(1, C // L):
            acc = acc + m[:, g * L:(g + 1) * L]
        return acc

    def count_le(t_f):
        def cbody(c, acc):
            d2 = d2_ref[:, pl.ds(c * C, C)]
            return acc + _lgroups(jnp.where(d2 <= t_f, 1.0, 0.0))
        acc = lax.fori_loop(c_lo, c_hi, cbody,
                            jnp.zeros((R, L), jnp.float32))
        return jnp.sum(acc, axis=1, keepdims=True)

    _CAP = 20

    def sbody(it, state):
        lo, hi, tau = state
        mid = lo + (hi - lo) // 2
        mid_f = lax.bitcast_convert_type(mid, jnp.float32)
        cnt = count_le(mid_f)
        done = jnp.logical_not(jnp.isinf(tau))
        hit = jnp.logical_and(cnt == float(_K), jnp.logical_not(done))
        tau = jnp.where(hit, mid_f, tau)
        take = cnt >= float(_K)
        lo = jnp.where(done, lo, jnp.where(take, lo, mid))
        hi = jnp.where(done, hi, jnp.where(take, mid, hi))
        return (lo, hi, tau)

    lo0 = jnp.full((R, 1), -1, jnp.int32)
    hi0 = jnp.full((R, 1), _INF_BITS, jnp.int32)
    tau0 = jnp.full((R, 1), jnp.inf, jnp.float32)
    lo, hi, tau = lax.fori_loop(0, _CAP, sbody, (lo0, hi0, tau0))
    hit_rows = jnp.logical_not(jnp.isinf(tau))
    a_f = jnp.where(hit_rows, tau,
                    lax.bitcast_convert_type(lo, jnp.float32))
    b_f = jnp.where(hit_rows, tau,
                    lax.bitcast_convert_type(hi, jnp.float32))

    def agg_body(c, carry):
        s_a, s_b, max_acc, n_a, n_b = carry
        d2 = d2_ref[:, pl.ds(c * C, C)]
        w = jnp.exp(-10.0 * d2)
        lt = d2 <= a_f
        le = d2 <= b_f
        h_c = h_ref[pl.ds(c * C, C), :]
        w_a = jnp.where(lt, w, 0.0)
        w_b = jnp.where(le, w, 0.0)
        s_a = s_a + jnp.dot(w_a, h_c, preferred_element_type=jnp.float32)
        s_b = s_b + jnp.dot(w_b, h_c, preferred_element_type=jnp.float32)
        n_a = n_a + _lgroups(jnp.where(lt, 1.0, 0.0))
        n_b = n_b + _lgroups(jnp.where(le, 1.0, 0.0))
        new_max = []
        for p in range(P):
            h_p = hT_ref[p:p + 1, pl.ds(c * C, C)]
            prod = jnp.where(le, w * h_p, -jnp.inf)
            m = jnp.maximum(prod[:, :L], prod[:, L:2 * L])
            for g in range(2, C // L):
                m = jnp.maximum(m, prod[:, g * L:(g + 1) * L])
            new_max.append(jnp.maximum(max_acc[p], m))
        return (s_a, s_b, tuple(new_max), n_a, n_b)

    zf = jnp.zeros((R, P), jnp.float32)
    zL = jnp.zeros((R, L), jnp.float32)
    max0 = tuple(jnp.full((R, L), -jnp.inf, jnp.float32) for _ in range(P))
    s_a, s_b, max_acc, n_a, n_b = lax.fori_loop(
        c_lo, c_hi, agg_body, (zf, zf, max0, zL, zL))

    n_a = jnp.sum(n_a, axis=1, keepdims=True)
    n_band = jnp.sum(n_b, axis=1, keepdims=True) - n_a
    frac = (float(_K) - n_a) / jnp.maximum(n_band, 1.0)
    maxs = jnp.concatenate(
        [jnp.max(m, axis=1, keepdims=True) for m in max_acc], axis=1)
    agg = jnp.concatenate([(s_a + frac * (s_b - s_a)) / float(_K), maxs],
                          axis=1)

    xe = xenc_ref[:, :]
    feats = (jnp.dot(xe, lo1_ref[:, :], preferred_element_type=jnp.float32)
             + jnp.dot(agg, lo2_ref[:, :], preferred_element_type=jnp.float32)
             + lo2b_ref[:, :])
    mu = jnp.mean(feats, axis=1, keepdims=True)
    d = feats - mu
    var = jnp.mean(d * d, axis=1, keepdims=True)
    f = d / jnp.sqrt(var + 1e-5)

    H = f.shape[1]
    W1s = f1_ref[:H, :] + f1_ref[H:, :]
    y = _elu(jnp.dot(f, W1s, preferred_element_type=jnp.float32) + f1b_ref[:, :])
    y = jnp.dot(y, f2_ref[:, :], preferred_element_type=jnp.float32) + f2b_ref[:, :]

    ncat = y.shape[1]
    z = (jnp.dot(y, o1_ref[:ncat, :], preferred_element_type=jnp.float32)
         + jnp.dot(xpfc_ref[:, :], o1_ref[ncat:, :],
                   preferred_element_type=jnp.float32)
         + o1b_ref[:, :])
    z = _elu(z)
    out_ref[:, :] = (jnp.dot(z, o2_ref[:, :], preferred_element_type=jnp.float32)
                     + o2b_ref[:, :])


def kernel(x_pfc, batch_pfc, enc_W1, enc_b1, enc_W2, enc_b2, enc_W3, enc_b3,
           lin_s_W, lin_s_b, lin_h_W, lin_h_b, lin_out1_W, lin_out2_W,
           lin_out2_b, ffn_W1, ffn_b1, ffn_W2, ffn_b2, out_W1, out_b1,
           out_W2, out_b2):
    N, PFC = x_pfc.shape
    S = lin_s_W.shape[1]
    P = lin_h_W.shape[1]
    H = enc_W3.shape[1]

    R = min(256, N)
    C = min(512, N)
    nblk = N // R

    row2 = lambda b: b.reshape(1, -1)

    xenc, s, h = pl.pallas_call(
        _enc_body,
        out_shape=(
            jax.ShapeDtypeStruct((N, H), jnp.float32),
            jax.ShapeDtypeStruct((N, S), jnp.float32),
            jax.ShapeDtypeStruct((N, P), jnp.float32),
        ),
    )(x_pfc, enc_W1, row2(enc_b1), enc_W2, row2(enc_b2), enc_W3, row2(enc_b3),
      lin_s_W, row2(lin_s_b), lin_h_W, row2(lin_h_b))

    sT = s.T
    hT = h.T
    bat = batch_pfc.astype(jnp.int32)
    bat_r = bat.reshape(N, 1)
    bat_c = bat.reshape(1, N)

    row_starts = jnp.arange(nblk, dtype=jnp.int32) * R
    b_lo = bat[row_starts]
    b_hi = bat[row_starts + (R - 1)]
    col_lo = jnp.searchsorted(bat, b_lo, side="left").astype(jnp.int32)
    col_hi = jnp.searchsorted(bat, b_hi, side="right").astype(jnp.int32)
    bounds = jnp.stack([col_lo // C, (col_hi + (C - 1)) // C], axis=1)

    full = lambda a: pl.BlockSpec(a.shape, lambda i, b: (0,) * a.ndim)
    rows = lambda ncol: pl.BlockSpec((R, ncol), lambda i, b: (i, 0))

    body = functools.partial(_grav_body, C)

    args = (s, xenc, x_pfc, bat_r, sT, hT, h, bat_c,
            lin_out1_W, lin_out2_W, row2(lin_out2_b),
            ffn_W1, row2(ffn_b1), ffn_W2, row2(ffn_b2),
            out_W1, row2(out_b1), out_W2, row2(out_b2))
    specs = [rows(S), rows(H), rows(PFC), rows(1),
             full(sT), full(hT), full(h), full(bat_c)] + \
            [full(a) for a in args[8:]]

    grid_spec = pltpu.PrefetchScalarGridSpec(
        num_scalar_prefetch=1,
        grid=(nblk,),
        in_specs=specs,
        out_specs=rows(1),
        scratch_shapes=[pltpu.VMEM((R, N), jnp.float32)],
    )

    out = pl.pallas_call(
        body,
        grid_spec=grid_spec,
        out_shape=jax.ShapeDtypeStruct((N, 1), jnp.float32),
        compiler_params=pltpu.CompilerParams(
            dimension_semantics=("arbitrary",)),
    )(bounds, *args)

    return (out, batch_pfc)


# speedup vs baseline: 1.3634x; 1.0622x over previous
import functools

import jax
import jax.numpy as jnp
from jax import lax
from jax.experimental import pallas as pl
from jax.experimental.pallas import tpu as pltpu

_K = 40
_INF_BITS = 0x7F800000


def _elu(x):
    return jnp.where(x > 0, x, jnp.exp(x) - 1.0)


def _enc_body(x_ref, W1_ref, b1_ref, W2_ref, b2_ref, W3_ref, b3_ref,
              Ws_ref, bs_ref, Wh_ref, bh_ref,
              xenc_ref, s_ref, h_ref):
    x = x_ref[:, :]
    a = _elu(jnp.dot(x, W1_ref[:, :], preferred_element_type=jnp.float32)
             + b1_ref[:, :])
    a = _elu(jnp.dot(a, W2_ref[:, :], preferred_element_type=jnp.float32)
             + b2_ref[:, :])
    xe = jnp.dot(a, W3_ref[:, :], preferred_element_type=jnp.float32) + b3_ref[:, :]
    xenc_ref[:, :] = xe
    s_ref[:, :] = jnp.dot(xe, Ws_ref[:, :], preferred_element_type=jnp.float32) + bs_ref[:, :]
    h_ref[:, :] = jnp.dot(xe, Wh_ref[:, :], preferred_element_type=jnp.float32) + bh_ref[:, :]


def _grav_body(C, bounds_ref, s_row_ref, xenc_ref, xpfc_ref, batr_ref,
               sT_ref, hT_ref, h_ref, batc_ref,
               lo1_ref, lo2_ref, lo2b_ref, f1_ref, f1b_ref, f2_ref, f2b_ref,
               o1_ref, o1b_ref, o2_ref, o2b_ref,
               out_ref, d2_ref):
    R = s_row_ref.shape[0]
    P = h_ref.shape[1]
    i = pl.program_id(0)
    c_lo = bounds_ref[i, 0]
    c_hi = bounds_ref[i, 1]

    s_r = s_row_ref[:, :]
    sq_r = jnp.sum(s_r * s_r, axis=1, keepdims=True)
    bat_r = batr_ref[:, :]

    def fill(c, _):
        sT_c = sT_ref[:, pl.ds(c * C, C)]
        sq_c = jnp.sum(sT_c * sT_c, axis=0, keepdims=True)
        d2 = sq_r + sq_c - 2.0 * jnp.dot(
            s_r, sT_c, preferred_element_type=jnp.float32)
        bat_c = batc_ref[:, pl.ds(c * C, C)]
        d2 = jnp.where(bat_r == bat_c, d2, jnp.inf)
        d2_ref[:, pl.ds(c * C, C)] = jnp.maximum(d2, 0.0)
        return 0

    lax.fori_loop(c_lo, c_hi, fill, 0)

    L = 128

    def _lgroups(m):
        acc = m[:, :L]
        for g in range(1, C // L):
            acc = acc + m[:, g * L:(g + 1) * L]
        return acc

    def count_le(t_f):
        def cbody(c, acc):
            d2 = d2_ref[:, pl.ds(c * C, C)]
            return acc + _lgroups(jnp.where(d2 <= t_f, 1.0, 0.0))
        acc = lax.fori_loop(c_lo, c_hi, cbody,
                            jnp.zeros((R, L), jnp.float32))
        return jnp.sum(acc, axis=1, keepdims=True)

    _CAP = 18

    def sbody(it, state):
        lo, hi, tau = state
        mid = lo + (hi - lo) // 2
        mid_f = lax.bitcast_convert_type(mid, jnp.float32)
        cnt = count_le(mid_f)
        done = jnp.logical_not(jnp.isinf(tau))
        hit = jnp.logical_and(cnt == float(_K), jnp.logical_not(done))
        tau = jnp.where(hit, mid_f, tau)
        take = cnt >= float(_K)
        lo = jnp.where(done, lo, jnp.where(take, lo, mid))
        hi = jnp.where(done, hi, jnp.where(take, mid, hi))
        return (lo, hi, tau)

    lo0 = jnp.full((R, 1), -1, jnp.int32)
    hi0 = jnp.full((R, 1), _INF_BITS, jnp.int32)
    tau0 = jnp.full((R, 1), jnp.inf, jnp.float32)
    lo, hi, tau = lax.fori_loop(0, _CAP, sbody, (lo0, hi0, tau0))
    hit_rows = jnp.logical_not(jnp.isinf(tau))
    a_f = jnp.where(hit_rows, tau,
                    lax.bitcast_convert_type(lo, jnp.float32))
    b_f = jnp.where(hit_rows, tau,
                    lax.bitcast_convert_type(hi, jnp.float32))

    def agg_body(c, carry):
        s_a, s_b, max_acc, n_a, n_b = carry
        d2 = d2_ref[:, pl.ds(c * C, C)]
        w = jnp.exp(-10.0 * d2)
        lt = d2 <= a_f
        le = d2 <= b_f
        h_c = h_ref[pl.ds(c * C, C), :]
        w_a = jnp.where(lt, w, 0.0)
        w_b = jnp.where(le, w, 0.0)
        s_a = s_a + jnp.dot(w_a, h_c, preferred_element_type=jnp.float32)
        s_b = s_b + jnp.dot(w_b, h_c, preferred_element_type=jnp.float32)
        n_a = n_a + _lgroups(jnp.where(lt, 1.0, 0.0))
        n_b = n_b + _lgroups(jnp.where(le, 1.0, 0.0))
        new_max = []
        for p in range(P):
            h_p = hT_ref[p:p + 1, pl.ds(c * C, C)]
            prod = jnp.where(le, w * h_p, -jnp.inf)
            m = jnp.maximum(prod[:, :L], prod[:, L:2 * L])
            for g in range(2, C // L):
                m = jnp.maximum(m, prod[:, g * L:(g + 1) * L])
            new_max.append(jnp.maximum(max_acc[p], m))
        return (s_a, s_b, tuple(new_max), n_a, n_b)

    zf = jnp.zeros((R, P), jnp.float32)
    zL = jnp.zeros((R, L), jnp.float32)
    max0 = tuple(jnp.full((R, L), -jnp.inf, jnp.float32) for _ in range(P))
    s_a, s_b, max_acc, n_a, n_b = lax.fori_loop(
        c_lo, c_hi, agg_body, (zf, zf, max0, zL, zL))

    n_a = jnp.sum(n_a, axis=1, keepdims=True)
    n_band = jnp.sum(n_b, axis=1, keepdims=True) - n_a
    frac = (float(_K) - n_a) / jnp.maximum(n_band, 1.0)
    maxs = jnp.concatenate(
        [jnp.max(m, axis=1, keepdims=True) for m in max_acc], axis=1)
    agg = jnp.concatenate([(s_a + frac * (s_b - s_a)) / float(_K), maxs],
                          axis=1)

    xe = xenc_ref[:, :]
    feats = (jnp.dot(xe, lo1_ref[:, :], preferred_element_type=jnp.float32)
             + jnp.dot(agg, lo2_ref[:, :], preferred_element_type=jnp.float32)
             + lo2b_ref[:, :])
    mu = jnp.mean(feats, axis=1, keepdims=True)
    d = feats - mu
    var = jnp.mean(d * d, axis=1, keepdims=True)
    f = d / jnp.sqrt(var + 1e-5)

    H = f.shape[1]
    W1s = f1_ref[:H, :] + f1_ref[H:, :]
    y = _elu(jnp.dot(f, W1s, preferred_element_type=jnp.float32) + f1b_ref[:, :])
    y = jnp.dot(y, f2_ref[:, :], preferred_element_type=jnp.float32) + f2b_ref[:, :]

    ncat = y.shape[1]
    z = (jnp.dot(y, o1_ref[:ncat, :], preferred_element_type=jnp.float32)
         + jnp.dot(xpfc_ref[:, :], o1_ref[ncat:, :],
                   preferred_element_type=jnp.float32)
         + o1b_ref[:, :])
    z = _elu(z)
    out_ref[:, :] = (jnp.dot(z, o2_ref[:, :], preferred_element_type=jnp.float32)
                     + o2b_ref[:, :])


def kernel(x_pfc, batch_pfc, enc_W1, enc_b1, enc_W2, enc_b2, enc_W3, enc_b3,
           lin_s_W, lin_s_b, lin_h_W, lin_h_b, lin_out1_W, lin_out2_W,
           lin_out2_b, ffn_W1, ffn_b1, ffn_W2, ffn_b2, out_W1, out_b1,
           out_W2, out_b2):
    N, PFC = x_pfc.shape
    S = lin_s_W.shape[1]
    P = lin_h_W.shape[1]
    H = enc_W3.shape[1]

    R = min(256, N)
    C = min(512, N)
    nblk = N // R

    row2 = lambda b: b.reshape(1, -1)

    xenc, s, h = pl.pallas_call(
        _enc_body,
        out_shape=(
            jax.ShapeDtypeStruct((N, H), jnp.float32),
            jax.ShapeDtypeStruct((N, S), jnp.float32),
            jax.ShapeDtypeStruct((N, P), jnp.float32),
        ),
    )(x_pfc, enc_W1, row2(enc_b1), enc_W2, row2(enc_b2), enc_W3, row2(enc_b3),
      lin_s_W, row2(lin_s_b), lin_h_W, row2(lin_h_b))

    sT = s.T
    hT = h.T
    bat = batch_pfc.astype(jnp.int32)
    bat_r = bat.reshape(N, 1)
    bat_c = bat.reshape(1, N)

    row_starts = jnp.arange(nblk, dtype=jnp.int32) * R
    b_lo = bat[row_starts]
    b_hi = bat[row_starts + (R - 1)]
    col_lo = jnp.searchsorted(bat, b_lo, side="left").astype(jnp.int32)
    col_hi = jnp.searchsorted(bat, b_hi, side="right").astype(jnp.int32)
    bounds = jnp.stack([col_lo // C, (col_hi + (C - 1)) // C], axis=1)

    full = lambda a: pl.BlockSpec(a.shape, lambda i, b: (0,) * a.ndim)
    rows = lambda ncol: pl.BlockSpec((R, ncol), lambda i, b: (i, 0))

    body = functools.partial(_grav_body, C)

    args = (s, xenc, x_pfc, bat_r, sT, hT, h, bat_c,
            lin_out1_W, lin_out2_W, row2(lin_out2_b),
            ffn_W1, row2(ffn_b1), ffn_W2, row2(ffn_b2),
            out_W1, row2(out_b1), out_W2, row2(out_b2))
    specs = [rows(S), rows(H), rows(PFC), rows(1),
             full(sT), full(hT), full(h), full(bat_c)] + \
            [full(a) for a in args[8:]]

    grid_spec = pltpu.PrefetchScalarGridSpec(
        num_scalar_prefetch=1,
        grid=(nblk,),
        in_specs=specs,
        out_specs=rows(1),
        scratch_shapes=[pltpu.VMEM((R, N), jnp.float32)],
    )

    out = pl.pallas_call(
        body,
        grid_spec=grid_spec,
        out_shape=jax.ShapeDtypeStruct((N, 1), jnp.float32),
        compiler_params=pltpu.CompilerParams(
            dimension_semantics=("arbitrary",)),
    )(bounds, *args)

    return (out, batch_pfc)
